# baseline (speedup 1.0000x reference)
import functools

import jax
import jax.numpy as jnp
from jax import lax
from jax.experimental import pallas as pl
from jax.experimental.pallas import tpu as pltpu
from jax.experimental.pallas import tpu_sc as plsc

N_CORES = 2
N_SUBCORES = 16
N_WORKERS = N_CORES * N_SUBCORES
CHUNK_ROWS = 64


def _sc_bcast(w_hbm, out_hbm, buf, sem):
    batch = out_hbm.shape[0]
    seq_len = out_hbm.shape[1]
    wid = lax.axis_index("s") * N_CORES + lax.axis_index("c")
    rows_per_w = seq_len // N_WORKERS
    n_chunks = rows_per_w // CHUNK_ROWS
    base0 = wid * rows_per_w

    def chunk_body(c, carry):
        base = base0 + c * CHUNK_ROWS
        pltpu.sync_copy(w_hbm.at[pl.ds(base, CHUNK_ROWS)], buf)
        for b in range(batch):
            pltpu.async_copy(buf, out_hbm.at[b, pl.ds(base, CHUNK_ROWS)], sem)
        for b in range(batch):
            pltpu.make_async_copy(buf, out_hbm.at[b, pl.ds(base, CHUNK_ROWS)], sem).wait()
        return carry

    lax.fori_loop(0, n_chunks, chunk_body, 0)


def kernel(tokens, W_pos):
    batch, seq_len = tokens.shape
    d = W_pos.shape[1]
    mesh = plsc.VectorSubcoreMesh(core_axis_name="c", subcore_axis_name="s")
    k = pl.kernel(
        _sc_bcast,
        mesh=mesh,
        out_type=jax.ShapeDtypeStruct((batch, seq_len, d), W_pos.dtype),
        scratch_types=[
            pltpu.VMEM((CHUNK_ROWS, d), W_pos.dtype),
            pltpu.SemaphoreType.DMA,
        ],
    )
    return k(W_pos[:seq_len])
